# baseline (speedup 1.0000x reference)
import jax
import jax.numpy as jnp
from jax import lax
from jax.experimental import pallas as pl
from jax.experimental.pallas import tpu as pltpu
from jax.experimental.pallas import tpu_sc as plsc

RES = 128
MARGIN = 0.1
NC = 2
NS = 16
NW = NC * NS
C = 1024
G = 256

_SC_PARAMS = pltpu.CompilerParams(needs_layout_passes=False,
                                  use_tc_tiling_on_sc=False)


def _minmax_body(px_ref, py_ref, pz_ref, o_ref):
    mins = jnp.stack([jnp.min(px_ref[...]), jnp.min(py_ref[...]),
                      jnp.min(pz_ref[...])])
    maxs = jnp.stack([jnp.max(px_ref[...]), jnp.max(py_ref[...]),
                      jnp.max(pz_ref[...])])
    o_ref[...] = jnp.stack([mins, maxs])


def _interleave_body(px_hbm, py_hbm, pz_hbm, tab_out,
                     pxv, pyv, pzv, rowv, sem):
    np_ = tab_out.shape[0]
    pw = rowv.shape[0]
    rows_per = np_ // NW
    wid = lax.axis_index("s") * NC + lax.axis_index("c")
    t = wid * rows_per
    bw = pl.multiple_of(jnp.minimum(t - lax.rem(t, 16), np_ - pw), 16)
    sl = pl.ds(bw, pw)
    pltpu.async_copy(px_hbm.at[sl], pxv, sem)
    pltpu.async_copy(py_hbm.at[sl], pyv, sem)
    pltpu.async_copy(pz_hbm.at[sl], pzv, sem)
    pltpu.make_async_copy(px_hbm.at[sl], pxv, sem).wait()
    pltpu.make_async_copy(py_hbm.at[sl], pyv, sem).wait()
    pltpu.make_async_copy(pz_hbm.at[sl], pzv, sem).wait()
    lane = lax.iota(jnp.int32, 16)
    zero = jnp.zeros((16,), jnp.int32)

    def grp(g, _):
        r = g * 16 + lane
        jsl = pl.ds(g * 16, 16)
        plsc.store_scatter(rowv, [r, zero], pxv[jsl])
        plsc.store_scatter(rowv, [r, zero + 1], pyv[jsl])
        plsc.store_scatter(rowv, [r, zero + 2], pzv[jsl])
        return 0

    lax.fori_loop(0, pw // 16, grp, 0, unroll=8)
    pltpu.sync_copy(rowv, tab_out.at[sl])


def _sc_body(qx_hbm, qy_hbm, qz_hbm, grid_hbm, tab_hbm,
             consts_hbm, dist_out, nn_out, *sc):
    buf = (dict(xs=sc[0], ys=sc[1], zs=sc[2], idx=sc[3], nn=sc[4],
                prow=sc[5], d=sc[6]),
           dict(xs=sc[7], ys=sc[8], zs=sc[9], idx=sc[10], nn=sc[11],
                prow=sc[12], d=sc[13]))
    consts_v = sc[14]
    sem_q = (sc[15], sc[16])
    sem_g = (sc[17], sc[18])
    sem_p = (sc[19], sc[20])
    sem_w = (sc[21], sc[22])

    nq = dist_out.shape[0]
    work = (nq // (16 * NW)) * 16
    nfull = work // C
    m = nfull + 1
    if m % 2 == 0:
        m += 1
    wid = lax.axis_index("s") * NC + lax.axis_index("c")
    base = wid * work
    end = jnp.where(wid == NW - 1, nq, base + work)

    def start(ci):
        return jnp.minimum(base + ci * C, end - C)

    pltpu.sync_copy(consts_hbm, consts_v)
    cv = consts_v[...]
    a0, a1, a2 = cv[0], cv[1], cv[2]
    b0, b1, b2 = cv[3], cv[4], cv[5]
    s = cv[6]

    def quant(v, a, b):
        t = jnp.minimum(jnp.maximum(v * a + b, 0.5), RES - 0.5)
        return t.astype(jnp.int32)

    def stage_a(ci, p):
        b = buf[p]
        sl = pl.ds(start(ci), C)
        pltpu.async_copy(qx_hbm.at[sl], b["xs"], sem_q[p])
        pltpu.async_copy(qy_hbm.at[sl], b["ys"], sem_q[p])
        pltpu.async_copy(qz_hbm.at[sl], b["zs"], sem_q[p])

    def stage_b(ci, p, drain_w):
        b = buf[p]
        if drain_w:
            osl = pl.ds(start(ci - 2), C)
            pltpu.make_async_copy(b["d"], dist_out.at[osl], sem_w[p]).wait()
            pltpu.make_async_copy(b["nn"], nn_out.at[osl], sem_w[p]).wait()
        sl = pl.ds(start(ci), C)
        pltpu.make_async_copy(qx_hbm.at[sl], b["xs"], sem_q[p]).wait()
        pltpu.make_async_copy(qy_hbm.at[sl], b["ys"], sem_q[p]).wait()
        pltpu.make_async_copy(qz_hbm.at[sl], b["zs"], sem_q[p]).wait()

        def grp(j, _):
            jsl = pl.ds(j * 16, 16)
            gx = quant(b["xs"][jsl], a0, b0)
            gy = quant(b["ys"][jsl], a1, b1)
            gz = quant(b["zs"][jsl], a2, b2)
            b["idx"][jsl] = (gx * RES + gy) * RES + gz
            return 0

        lax.fori_loop(0, C // 16, grp, 0, unroll=8)
        for k in range(C // G):
            ksl = pl.ds(k * G, G)
            pltpu.async_copy(grid_hbm.at[b["idx"].at[ksl]],
                             b["nn"].at[ksl], sem_g[p])

    def stage_c(ci, p):
        b = buf[p]
        for k in range(C // G):
            ksl = pl.ds(k * G, G)
            pltpu.make_async_copy(grid_hbm.at[b["idx"].at[ksl]],
                                  b["nn"].at[ksl], sem_g[p]).wait()
        for k in range(C // G):
            ksl = pl.ds(k * G, G)
            pltpu.async_copy(tab_hbm.at[b["nn"].at[ksl]],
                             b["prow"].at[ksl], sem_p[p])

    def stage_d(ci, p):
        b = buf[p]
        for k in range(C // G):
            ksl = pl.ds(k * G, G)
            pltpu.make_async_copy(tab_hbm.at[b["nn"].at[ksl]],
                                  b["prow"].at[ksl], sem_p[p]).wait()
        lane = lax.iota(jnp.int32, 16)
        zero = jnp.zeros((16,), jnp.int32)

        def grp2(j, _):
            jsl = pl.ds(j * 16, 16)
            r = j * 16 + lane
            dx = plsc.load_gather(b["prow"], [r, zero]) * s - b["xs"][jsl]
            dy = plsc.load_gather(b["prow"], [r, zero + 1]) * s - b["ys"][jsl]
            dz = plsc.load_gather(b["prow"], [r, zero + 2]) * s - b["zs"][jsl]
            d2 = dx * dx + dy * dy + dz * dz
            yi = jnp.int32(0x5F3759DF) - lax.shift_right_logical(
                plsc.bitcast(d2, jnp.int32), 1)
            y = plsc.bitcast(yi, jnp.float32)
            hd = 0.5 * d2
            y = y * (1.5 - hd * y * y)
            y = y * (1.5 - hd * y * y)
            b["d"][jsl] = d2 * y
            return 0

        lax.fori_loop(0, C // 16, grp2, 0, unroll=8)
        osl = pl.ds(start(ci), C)
        pltpu.async_copy(b["d"], dist_out.at[osl], sem_w[p])
        pltpu.async_copy(b["nn"], nn_out.at[osl], sem_w[p])

    stage_a(0, 0)
    stage_a(1, 1)
    stage_b(0, 0, False)

    def pair_steps(i0, first):
        stage_c(i0, 0)
        stage_b(i0 + 1, 1, not first)
        stage_d(i0, 0)
        stage_a(i0 + 2, 0)
        stage_c(i0 + 1, 1)
        stage_b(i0 + 2, 0, True)
        stage_d(i0 + 1, 1)
        stage_a(i0 + 3, 1)

    pair_steps(0, True)

    def pair_body(pair, _):
        pair_steps(2 * pair, False)
        return 0

    lax.fori_loop(1, (m - 1) // 2, pair_body, 0)
    stage_c(m - 1, 0)
    stage_d(m - 1, 0)
    sl = pl.ds(start(m), C)
    pltpu.make_async_copy(qx_hbm.at[sl], buf[1]["xs"], sem_q[1]).wait()
    pltpu.make_async_copy(qy_hbm.at[sl], buf[1]["ys"], sem_q[1]).wait()
    pltpu.make_async_copy(qz_hbm.at[sl], buf[1]["zs"], sem_q[1]).wait()
    osl0 = pl.ds(start(m - 1), C)
    pltpu.make_async_copy(buf[0]["d"], dist_out.at[osl0], sem_w[0]).wait()
    pltpu.make_async_copy(buf[0]["nn"], nn_out.at[osl0], sem_w[0]).wait()
    osl1 = pl.ds(start(m - 2), C)
    pltpu.make_async_copy(buf[1]["d"], dist_out.at[osl1], sem_w[1]).wait()
    pltpu.make_async_copy(buf[1]["nn"], nn_out.at[osl1], sem_w[1]).wait()


def kernel(query_points, scale, points, nn_idxs_grid):
    nq = query_points.shape[0]
    npts = points.shape[0]

    qx = query_points[:, 0]
    qy = query_points[:, 1]
    qz = query_points[:, 2]
    px = points[:, 0]
    py = points[:, 1]
    pz = points[:, 2]
    grid_flat = nn_idxs_grid.reshape(-1)

    mm = pl.pallas_call(
        _minmax_body,
        out_shape=jax.ShapeDtypeStruct((2, 3), jnp.float32),
    )(px, py, pz)
    lb = mm[0]
    ub = mm[1]
    ext = ub - lb
    lower = lb - MARGIN * ext
    upper = ub + MARGIN * ext
    fac = (RES - 1.0) / (upper - lower)
    inv_s = 1.0 / scale
    qa = inv_s * fac
    qb = 0.5 - lower * fac
    consts = jnp.concatenate(
        [qa, qb, jnp.stack([scale]), jnp.zeros((9,), jnp.float32)])

    mesh = plsc.VectorSubcoreMesh(core_axis_name="c", subcore_axis_name="s",
                                  num_cores=NC, num_subcores=NS)

    pw = ((npts // NW + 16 + 15) // 16) * 16
    interleave = pl.kernel(
        _interleave_body,
        out_type=jax.ShapeDtypeStruct((npts, 4), jnp.float32),
        mesh=mesh,
        compiler_params=_SC_PARAMS,
        scratch_types=[
            pltpu.VMEM((pw,), jnp.float32),
            pltpu.VMEM((pw,), jnp.float32),
            pltpu.VMEM((pw,), jnp.float32),
            pltpu.VMEM((pw, 4), jnp.float32),
            pltpu.SemaphoreType.DMA,
        ],
    )
    tab = interleave(px, py, pz)

    fbuf = [pltpu.VMEM((C,), jnp.float32)] * 3 \
        + [pltpu.VMEM((C,), jnp.int32)] * 2 \
        + [pltpu.VMEM((C, 4), jnp.float32)] \
        + [pltpu.VMEM((C,), jnp.float32)]
    sc = pl.kernel(
        _sc_body,
        out_type=(jax.ShapeDtypeStruct((nq,), jnp.float32),
                  jax.ShapeDtypeStruct((nq,), jnp.int32)),
        mesh=mesh,
        compiler_params=_SC_PARAMS,
        scratch_types=fbuf + fbuf + [pltpu.VMEM((16,), jnp.float32)]
        + [pltpu.SemaphoreType.DMA] * 8,
    )
    dist, nn = sc(qx, qy, qz, grid_flat, tab, consts)
    return (dist, nn)


# speedup vs baseline: 1.0153x; 1.0153x over previous
import jax
import jax.numpy as jnp
from jax import lax
from jax.experimental import pallas as pl
from jax.experimental.pallas import tpu as pltpu
from jax.experimental.pallas import tpu_sc as plsc

RES = 128
MARGIN = 0.1
NC = 2
NS = 16
NW = NC * NS
C = 1024
G = 128

_SC_PARAMS = pltpu.CompilerParams(needs_layout_passes=False,
                                  use_tc_tiling_on_sc=False)


def _interleave_body(px_hbm, py_hbm, pz_hbm, tab_out, part_out,
                     pxv, pyv, pzv, rowv, partv, sem):
    np_ = tab_out.shape[0]
    pw = rowv.shape[0]
    rows_per = np_ // NW
    wid = lax.axis_index("s") * NC + lax.axis_index("c")
    t = wid * rows_per
    bw = pl.multiple_of(jnp.minimum(t - lax.rem(t, 16), np_ - pw), 16)
    sl = pl.ds(bw, pw)
    pltpu.async_copy(px_hbm.at[sl], pxv, sem)
    pltpu.async_copy(py_hbm.at[sl], pyv, sem)
    pltpu.async_copy(pz_hbm.at[sl], pzv, sem)
    pltpu.make_async_copy(px_hbm.at[sl], pxv, sem).wait()
    pltpu.make_async_copy(py_hbm.at[sl], pyv, sem).wait()
    pltpu.make_async_copy(pz_hbm.at[sl], pzv, sem).wait()
    lane = lax.iota(jnp.int32, 16)
    zero = jnp.zeros((16,), jnp.int32)

    def grp(g, acc):
        mnx, mny, mnz, mxx, mxy, mxz = acc
        r = g * 16 + lane
        jsl = pl.ds(g * 16, 16)
        vx = pxv[jsl]
        vy = pyv[jsl]
        vz = pzv[jsl]
        plsc.store_scatter(rowv, [r, zero], vx)
        plsc.store_scatter(rowv, [r, zero + 1], vy)
        plsc.store_scatter(rowv, [r, zero + 2], vz)
        return (jnp.minimum(mnx, vx), jnp.minimum(mny, vy),
                jnp.minimum(mnz, vz), jnp.maximum(mxx, vx),
                jnp.maximum(mxy, vy), jnp.maximum(mxz, vz))

    big = jnp.full((16,), 3.0e38, jnp.float32)
    acc = lax.fori_loop(0, pw // 16, grp,
                        (big, big, big, -big, -big, -big), unroll=8)
    for i, v in enumerate(acc):
        partv[pl.ds(i * 16, 16)] = v
    pltpu.sync_copy(rowv, tab_out.at[sl])
    pltpu.sync_copy(partv, part_out.at[pl.ds(wid * 96, 96)])


def _sc_body(qx_hbm, qy_hbm, qz_hbm, grid_hbm, tab_hbm,
             part_hbm, scale_hbm, dist_out, nn_out, *sc):
    buf = (dict(xs=sc[0], ys=sc[1], zs=sc[2], idx=sc[3], nn=sc[4],
                prow=sc[5], d=sc[6]),
           dict(xs=sc[7], ys=sc[8], zs=sc[9], idx=sc[10], nn=sc[11],
                prow=sc[12], d=sc[13]))
    partv = sc[14]
    scalev = sc[15]
    sem_q = (sc[16], sc[17])
    sem_g = (sc[18], sc[19])
    sem_p = (sc[20], sc[21])
    sem_w = (sc[22], sc[23])

    nq = dist_out.shape[0]
    work = (nq // (16 * NW)) * 16
    nfull = work // C
    m = nfull + 1
    if m % 2 == 0:
        m += 1
    wid = lax.axis_index("s") * NC + lax.axis_index("c")
    base = wid * work
    end = jnp.where(wid == NW - 1, nq, base + work)

    def start(ci):
        return jnp.minimum(base + ci * C, end - C)

    pltpu.sync_copy(part_hbm, partv)
    pltpu.sync_copy(scale_hbm, scalev)
    s = scalev[...]
    inv_s = 1.0 / s

    def comb(w, acc):
        o = w * 96
        return (jnp.minimum(acc[0], partv[pl.ds(o, 16)]),
                jnp.minimum(acc[1], partv[pl.ds(o + 16, 16)]),
                jnp.minimum(acc[2], partv[pl.ds(o + 32, 16)]),
                jnp.maximum(acc[3], partv[pl.ds(o + 48, 16)]),
                jnp.maximum(acc[4], partv[pl.ds(o + 64, 16)]),
                jnp.maximum(acc[5], partv[pl.ds(o + 80, 16)]))

    big = jnp.full((16,), 3.0e38, jnp.float32)
    acc = lax.fori_loop(0, NW, comb, (big, big, big, -big, -big, -big),
                        unroll=8)
    def ab(mnv, mxv):
        lb = jnp.broadcast_to(jnp.min(mnv), (16,))
        ub = jnp.broadcast_to(jnp.max(mxv), (16,))
        ext = ub - lb
        lower = lb - MARGIN * ext
        upper = ub + MARGIN * ext
        fac = (RES - 1.0) / (upper - lower)
        return inv_s * fac, 0.5 - lower * fac

    a0, b0 = ab(acc[0], acc[3])
    a1, b1 = ab(acc[1], acc[4])
    a2, b2 = ab(acc[2], acc[5])

    def quant(v, a, b):
        t = jnp.minimum(jnp.maximum(v * a + b, 0.5), RES - 0.5)
        return t.astype(jnp.int32)

    def stage_a(ci, p):
        b = buf[p]
        sl = pl.ds(start(ci), C)
        pltpu.async_copy(qx_hbm.at[sl], b["xs"], sem_q[p])
        pltpu.async_copy(qy_hbm.at[sl], b["ys"], sem_q[p])
        pltpu.async_copy(qz_hbm.at[sl], b["zs"], sem_q[p])

    def stage_b(ci, p, drain_w):
        b = buf[p]
        if drain_w:
            osl = pl.ds(start(ci - 2), C)
            pltpu.make_async_copy(b["d"], dist_out.at[osl], sem_w[p]).wait()
            pltpu.make_async_copy(b["nn"], nn_out.at[osl], sem_w[p]).wait()
        sl = pl.ds(start(ci), C)
        pltpu.make_async_copy(qx_hbm.at[sl], b["xs"], sem_q[p]).wait()
        pltpu.make_async_copy(qy_hbm.at[sl], b["ys"], sem_q[p]).wait()
        pltpu.make_async_copy(qz_hbm.at[sl], b["zs"], sem_q[p]).wait()

        def grp(j, _):
            jsl = pl.ds(j * 16, 16)
            gx = quant(b["xs"][jsl], a0, b0)
            gy = quant(b["ys"][jsl], a1, b1)
            gz = quant(b["zs"][jsl], a2, b2)
            b["idx"][jsl] = (gx * RES + gy) * RES + gz
            return 0

        lax.fori_loop(0, C // 16, grp, 0, unroll=8)
        for k in range(C // G):
            ksl = pl.ds(k * G, G)
            pltpu.async_copy(grid_hbm.at[b["idx"].at[ksl]],
                             b["nn"].at[ksl], sem_g[p])

    def stage_c(ci, p):
        b = buf[p]
        for k in range(C // G):
            ksl = pl.ds(k * G, G)
            pltpu.make_async_copy(grid_hbm.at[b["idx"].at[ksl]],
                                  b["nn"].at[ksl], sem_g[p]).wait()
        for k in range(C // G):
            ksl = pl.ds(k * G, G)
            pltpu.async_copy(tab_hbm.at[b["nn"].at[ksl]],
                             b["prow"].at[ksl], sem_p[p])

    def stage_d(ci, p):
        b = buf[p]
        for k in range(C // G):
            ksl = pl.ds(k * G, G)
            pltpu.make_async_copy(tab_hbm.at[b["nn"].at[ksl]],
                                  b["prow"].at[ksl], sem_p[p]).wait()
        lane = lax.iota(jnp.int32, 16)
        zero = jnp.zeros((16,), jnp.int32)

        def grp2(j, _):
            jsl = pl.ds(j * 16, 16)
            r = j * 16 + lane
            dx = plsc.load_gather(b["prow"], [r, zero]) * s - b["xs"][jsl]
            dy = plsc.load_gather(b["prow"], [r, zero + 1]) * s - b["ys"][jsl]
            dz = plsc.load_gather(b["prow"], [r, zero + 2]) * s - b["zs"][jsl]
            d2 = dx * dx + dy * dy + dz * dz
            yi = jnp.int32(0x5F3759DF) - lax.shift_right_logical(
                plsc.bitcast(d2, jnp.int32), 1)
            y = plsc.bitcast(yi, jnp.float32)
            hd = 0.5 * d2
            y = y * (1.5 - hd * y * y)
            y = y * (1.5 - hd * y * y)
            b["d"][jsl] = d2 * y
            return 0

        lax.fori_loop(0, C // 16, grp2, 0, unroll=8)
        osl = pl.ds(start(ci), C)
        pltpu.async_copy(b["d"], dist_out.at[osl], sem_w[p])
        pltpu.async_copy(b["nn"], nn_out.at[osl], sem_w[p])

    stage_a(0, 0)
    stage_a(1, 1)
    stage_b(0, 0, False)

    def pair_steps(i0, first):
        stage_c(i0, 0)
        stage_b(i0 + 1, 1, not first)
        stage_d(i0, 0)
        stage_a(i0 + 2, 0)
        stage_c(i0 + 1, 1)
        stage_b(i0 + 2, 0, True)
        stage_d(i0 + 1, 1)
        stage_a(i0 + 3, 1)

    pair_steps(0, True)

    def pair_body(pair, _):
        pair_steps(2 * pair, False)
        return 0

    lax.fori_loop(1, (m - 1) // 2, pair_body, 0)
    stage_c(m - 1, 0)
    stage_d(m - 1, 0)
    sl = pl.ds(start(m), C)
    pltpu.make_async_copy(qx_hbm.at[sl], buf[1]["xs"], sem_q[1]).wait()
    pltpu.make_async_copy(qy_hbm.at[sl], buf[1]["ys"], sem_q[1]).wait()
    pltpu.make_async_copy(qz_hbm.at[sl], buf[1]["zs"], sem_q[1]).wait()
    osl0 = pl.ds(start(m - 1), C)
    pltpu.make_async_copy(buf[0]["d"], dist_out.at[osl0], sem_w[0]).wait()
    pltpu.make_async_copy(buf[0]["nn"], nn_out.at[osl0], sem_w[0]).wait()
    osl1 = pl.ds(start(m - 2), C)
    pltpu.make_async_copy(buf[1]["d"], dist_out.at[osl1], sem_w[1]).wait()
    pltpu.make_async_copy(buf[1]["nn"], nn_out.at[osl1], sem_w[1]).wait()


def kernel(query_points, scale, points, nn_idxs_grid):
    nq = query_points.shape[0]
    npts = points.shape[0]

    qx = query_points[:, 0]
    qy = query_points[:, 1]
    qz = query_points[:, 2]
    px = points[:, 0]
    py = points[:, 1]
    pz = points[:, 2]
    grid_flat = nn_idxs_grid.reshape(-1)

    scale_vec = jnp.full((16,), scale, jnp.float32)

    mesh = plsc.VectorSubcoreMesh(core_axis_name="c", subcore_axis_name="s",
                                  num_cores=NC, num_subcores=NS)

    pw = ((npts // NW + 16 + 15) // 16) * 16
    interleave = pl.kernel(
        _interleave_body,
        out_type=(jax.ShapeDtypeStruct((npts, 4), jnp.float32),
                  jax.ShapeDtypeStruct((NW * 96,), jnp.float32)),
        mesh=mesh,
        compiler_params=_SC_PARAMS,
        scratch_types=[
            pltpu.VMEM((pw,), jnp.float32),
            pltpu.VMEM((pw,), jnp.float32),
            pltpu.VMEM((pw,), jnp.float32),
            pltpu.VMEM((pw, 4), jnp.float32),
            pltpu.VMEM((96,), jnp.float32),
            pltpu.SemaphoreType.DMA,
        ],
    )
    tab, part = interleave(px, py, pz)

    fbuf = [pltpu.VMEM((C,), jnp.float32)] * 3 \
        + [pltpu.VMEM((C,), jnp.int32)] * 2 \
        + [pltpu.VMEM((C, 4), jnp.float32)] \
        + [pltpu.VMEM((C,), jnp.float32)]
    sc = pl.kernel(
        _sc_body,
        out_type=(jax.ShapeDtypeStruct((nq,), jnp.float32),
                  jax.ShapeDtypeStruct((nq,), jnp.int32)),
        mesh=mesh,
        compiler_params=_SC_PARAMS,
        scratch_types=fbuf + fbuf
        + [pltpu.VMEM((NW * 96,), jnp.float32),
           pltpu.VMEM((16,), jnp.float32)]
        + [pltpu.SemaphoreType.DMA] * 8,
    )
    dist, nn = sc(qx, qy, qz, grid_flat, tab, part, scale_vec)
    return (dist, nn)
